# Initial kernel scaffold; baseline (speedup 1.0000x reference)
import functools

import jax
import jax.numpy as jnp
from jax import lax
from jax.experimental import pallas as pl
from jax.experimental.pallas import tpu as pltpu
from jax.experimental.pallas import tpu_sc as plsc

N = 10000
E = 320000
F = 128
HEADS = 8
HIDDEN = 16
AW = 16

NC, NS = 2, 16
NW = NC * NS
CHUNK = 128
NCHUNK = E // CHUNK
ROWS_PER_SUB = N // NS

_SC_MESH = plsc.VectorSubcoreMesh(
    core_axis_name="c", subcore_axis_name="s", num_cores=NC, num_subcores=NS
)


def _edge_body(src_hbm, dst_hbm, h_hbm, as_hbm, ad_hbm, zm_hbm, zd_hbm,
               outm_hbm, outd_hbm,
               srcv, dstv, hbuf, asbuf, adbuf, exbuf,
               accm, accd, sem0, sem1, sem2, sem3, sem4):
    c = lax.axis_index("c")
    s = lax.axis_index("s")
    gwid = s * NC + c

    r0 = s * ROWS_PER_SUB
    pltpu.sync_copy(zm_hbm.at[pl.ds(r0, ROWS_PER_SUB)],
                    accm.at[pl.ds(r0, ROWS_PER_SUB)])
    pltpu.sync_copy(zd_hbm.at[pl.ds(r0, ROWS_PER_SUB)],
                    accd.at[pl.ds(r0, ROWS_PER_SUB)])
    plsc.subcore_barrier()

    nq_lo = NCHUNK // NW
    nq = jnp.where(gwid < (NCHUNK - nq_lo * NW), nq_lo + 1, nq_lo)

    def chunk_body(i, carry):
        base = (gwid + i * NW) * CHUNK
        cp0 = pltpu.async_copy(src_hbm.at[pl.ds(base, CHUNK)], srcv, sem0)
        cp1 = pltpu.async_copy(dst_hbm.at[pl.ds(base, CHUNK)], dstv, sem1)
        cp0.wait()
        cp1.wait()
        g0 = pltpu.async_copy(h_hbm.at[srcv], hbuf, sem2)
        g1 = pltpu.async_copy(as_hbm.at[srcv], asbuf, sem3)
        g2 = pltpu.async_copy(ad_hbm.at[dstv], adbuf, sem4)
        g1.wait()
        g2.wait()

        def edge_body(e, carry2):
            a = asbuf[e, :] + adbuf[e, :]
            ex = jnp.exp(jnp.maximum(a, 0.2 * a))
            exbuf[e, :] = ex
            return carry2

        lax.fori_loop(0, CHUNK, edge_body, 0)
        g0.wait()

        def scale_body(e, carry2):
            row_i = jnp.full((16,), e, jnp.int32)
            for g in range(HEADS):
                w = plsc.load_gather(exbuf, [row_i, jnp.full((16,), g, jnp.int32)])
                hv = hbuf[e, pl.ds(g * HIDDEN, HIDDEN)]
                hbuf[e, pl.ds(g * HIDDEN, HIDDEN)] = hv * w
            return carry2

        lax.fori_loop(0, CHUNK, scale_body, 0)

        pltpu.sync_copy(hbuf, accm.at[dstv], add=True)
        pltpu.sync_copy(exbuf, accd.at[dstv], add=True)
        return carry

    lax.fori_loop(0, nq, chunk_body, 0)

    plsc.subcore_barrier()
    pltpu.sync_copy(accm.at[pl.ds(r0, ROWS_PER_SUB)],
                    outm_hbm.at[c, pl.ds(r0, ROWS_PER_SUB)])
    pltpu.sync_copy(accd.at[pl.ds(r0, ROWS_PER_SUB)],
                    outd_hbm.at[c, pl.ds(r0, ROWS_PER_SUB)])


_sc_edge = functools.partial(
    pl.kernel,
    out_type=(
        jax.ShapeDtypeStruct((NC, N, F), jnp.float32),
        jax.ShapeDtypeStruct((NC, N, AW), jnp.float32),
    ),
    mesh=_SC_MESH,
    scratch_types=[
        pltpu.VMEM((CHUNK,), jnp.int32),
        pltpu.VMEM((CHUNK,), jnp.int32),
        pltpu.VMEM((CHUNK, F), jnp.float32),
        pltpu.VMEM((CHUNK, AW), jnp.float32),
        pltpu.VMEM((CHUNK, AW), jnp.float32),
        pltpu.VMEM((CHUNK, AW), jnp.float32),
        pltpu.VMEM_SHARED((N, F), jnp.float32),
        pltpu.VMEM_SHARED((N, AW), jnp.float32),
        pltpu.SemaphoreType.DMA,
        pltpu.SemaphoreType.DMA,
        pltpu.SemaphoreType.DMA,
        pltpu.SemaphoreType.DMA,
        pltpu.SemaphoreType.DMA,
    ],
)(_edge_body)


_BLK = 1000
_GRID = N // _BLK


def _tables_body(x_ref, wh_ref, was_ref, wad_ref, h_ref, as_ref, ad_ref):
    x = x_ref[...]
    h_ref[...] = jnp.dot(x, wh_ref[...], preferred_element_type=jnp.float32)
    as_ref[...] = jnp.dot(x, was_ref[...], preferred_element_type=jnp.float32)
    ad_ref[...] = jnp.dot(x, wad_ref[...], preferred_element_type=jnp.float32)


def _tc_tables(x, wh, was, wad):
    return pl.pallas_call(
        _tables_body,
        grid=(_GRID,),
        in_specs=[
            pl.BlockSpec((_BLK, F), lambda i: (i, 0)),
            pl.BlockSpec((F, F), lambda i: (0, 0)),
            pl.BlockSpec((F, AW), lambda i: (0, 0)),
            pl.BlockSpec((F, AW), lambda i: (0, 0)),
        ],
        out_specs=[
            pl.BlockSpec((_BLK, F), lambda i: (i, 0)),
            pl.BlockSpec((_BLK, AW), lambda i: (i, 0)),
            pl.BlockSpec((_BLK, AW), lambda i: (i, 0)),
        ],
        out_shape=[
            jax.ShapeDtypeStruct((N, F), jnp.float32),
            jax.ShapeDtypeStruct((N, AW), jnp.float32),
            jax.ShapeDtypeStruct((N, AW), jnp.float32),
        ],
    )(x, wh, was, wad)


def _finalize(pm_ref, pd_ref, b_ref, r_ref):
    m = pm_ref[0] + pm_ref[1]
    dsum = pd_ref[0] + pd_ref[1]
    rec = 1.0 / (dsum + 1e-16)
    recb = jnp.dot(rec, r_ref[...], preferred_element_type=jnp.float32)
    h = m * recb + b_ref[...]
    return jnp.where(h > 0, h, jnp.exp(h) - 1.0)


def _combine_body(pm_ref, pd_ref, b_ref, r_ref, wh_ref, was_ref, wad_ref,
                  h_ref, as_ref, ad_ref):
    act = _finalize(pm_ref, pd_ref, b_ref, r_ref)
    h_ref[...] = jnp.dot(act, wh_ref[...], preferred_element_type=jnp.float32)
    as_ref[...] = jnp.dot(act, was_ref[...], preferred_element_type=jnp.float32)
    ad_ref[...] = jnp.dot(act, wad_ref[...], preferred_element_type=jnp.float32)


def _tc_combine(pm, pd, b, r, wh, was, wad):
    return pl.pallas_call(
        _combine_body,
        grid=(_GRID,),
        in_specs=[
            pl.BlockSpec((NC, _BLK, F), lambda i: (0, i, 0)),
            pl.BlockSpec((NC, _BLK, AW), lambda i: (0, i, 0)),
            pl.BlockSpec((1, F), lambda i: (0, 0)),
            pl.BlockSpec((AW, F), lambda i: (0, 0)),
            pl.BlockSpec((F, F), lambda i: (0, 0)),
            pl.BlockSpec((F, AW), lambda i: (0, 0)),
            pl.BlockSpec((F, AW), lambda i: (0, 0)),
        ],
        out_specs=[
            pl.BlockSpec((_BLK, F), lambda i: (i, 0)),
            pl.BlockSpec((_BLK, AW), lambda i: (i, 0)),
            pl.BlockSpec((_BLK, AW), lambda i: (i, 0)),
        ],
        out_shape=[
            jax.ShapeDtypeStruct((N, F), jnp.float32),
            jax.ShapeDtypeStruct((N, AW), jnp.float32),
            jax.ShapeDtypeStruct((N, AW), jnp.float32),
        ],
    )(pm, pd, b, r, wh, was, wad)


def _head_body(pm_ref, pd_ref, b_ref, r_ref, lw_ref, extra_ref, out_ref):
    act = _finalize(pm_ref, pd_ref, b_ref, r_ref)
    colsum = jnp.sum(act, axis=0)
    val = jnp.sum(colsum * lw_ref[0]) * (1.0 / N) + extra_ref[0, 0]
    out_ref[...] = jnp.reshape(val, (1, 1))


def _tc_head(pm, pd, b, r, lw, extra):
    return pl.pallas_call(
        _head_body,
        out_shape=jax.ShapeDtypeStruct((1, 1), jnp.float32),
    )(pm, pd, b, r, lw, extra)


def _fold_att(att):
    m = jnp.zeros((F, AW), jnp.float32)
    return m.at[jnp.arange(F), jnp.arange(F) // HIDDEN].set(att.reshape(F))


def kernel(x, edge_index, u, w, W1, att_src1, att_dst1, b1,
           W2, att_src2, att_dst2, b2, lin_W, lin_b):
    src = edge_index[0]
    dst = edge_index[1]

    was1 = W1 @ _fold_att(att_src1)
    wad1 = W1 @ _fold_att(att_dst1)
    was2 = W2 @ _fold_att(att_src2)
    wad2 = W2 @ _fold_att(att_dst2)
    rmat = (jnp.arange(F)[None, :] // HIDDEN
            == jnp.arange(AW)[:, None]).astype(jnp.float32)
    zm = jnp.zeros((N, F), jnp.float32)
    zd = jnp.zeros((N, AW), jnp.float32)
    b1r = b1.reshape(1, F)
    b2r = b2.reshape(1, F)
    lw = lin_W[:F, 0].reshape(1, F)
    extra = (u * lin_W[F, 0] + w * lin_W[F + 1, 0] + lin_b[0]).astype(
        jnp.float32).reshape(1, 1)

    h1, as1, ad1 = _tc_tables(x, W1, was1, wad1)
    pm1, pd1 = _sc_edge(src, dst, h1, as1, ad1, zm, zd)
    h2, as2, ad2 = _tc_combine(pm1, pd1, b1r, rmat, W2, was2, wad2)
    pm2, pd2 = _sc_edge(src, dst, h2, as2, ad2, zm, zd)
    out = _tc_head(pm2, pd2, b2r, rmat, lw, extra)
    return out.reshape(1)


# speedup vs baseline: 56.7636x; 56.7636x over previous
import functools

import jax
import jax.numpy as jnp
from jax import lax
from jax.experimental import pallas as pl
from jax.experimental.pallas import tpu as pltpu
from jax.experimental.pallas import tpu_sc as plsc

N = 10000
E = 320000
F = 128
HEADS = 8
HIDDEN = 16
AW = 16

NC, NS = 2, 16
NW = NC * NS
CHUNK = 128
NCHUNK = E // CHUNK
NPAD = 10240
ROWS_PER_SUB = NPAD // NS

_SC_MESH = plsc.VectorSubcoreMesh(
    core_axis_name="c", subcore_axis_name="s", num_cores=NC, num_subcores=NS
)


def _edge_body(src_hbm, dst_hbm, h_hbm, as_hbm, ad_hbm, zm_hbm, zd_hbm,
               outm_hbm, outd_hbm,
               srcv, dstv, hbuf, asbuf, adbuf, exbuf,
               accm, accd, sem0, sem1, sem2, sem3, sem4):
    c = lax.axis_index("c")
    s = lax.axis_index("s")
    gwid = s * NC + c

    r0 = s * ROWS_PER_SUB
    pltpu.sync_copy(zm_hbm.at[pl.ds(r0, ROWS_PER_SUB)],
                    accm.at[pl.ds(r0, ROWS_PER_SUB)])
    pltpu.sync_copy(zd_hbm.at[pl.ds(r0, ROWS_PER_SUB)],
                    accd.at[pl.ds(r0, ROWS_PER_SUB)])
    plsc.subcore_barrier()

    nq_lo = NCHUNK // NW
    nq = jnp.where(gwid < (NCHUNK - nq_lo * NW), nq_lo + 1, nq_lo)

    def chunk_body(i, carry):
        base = (gwid + i * NW) * CHUNK
        cp0 = pltpu.async_copy(src_hbm.at[pl.ds(base, CHUNK)], srcv, sem0)
        cp1 = pltpu.async_copy(dst_hbm.at[pl.ds(base, CHUNK)], dstv, sem1)
        cp0.wait()
        cp1.wait()
        g0 = pltpu.async_copy(h_hbm.at[srcv], hbuf, sem2)
        g1 = pltpu.async_copy(as_hbm.at[srcv], asbuf, sem3)
        g2 = pltpu.async_copy(ad_hbm.at[dstv], adbuf, sem4)
        g1.wait()
        g2.wait()

        def edge_body(e, carry2):
            a = asbuf[e, :] + adbuf[e, :]
            ex = jnp.exp(jnp.maximum(a, 0.2 * a))
            exbuf[e, :] = ex
            return carry2

        lax.fori_loop(0, CHUNK, edge_body, 0)
        g0.wait()

        def scale_body(e, carry2):
            row_i = jnp.full((16,), e, jnp.int32)
            for g in range(HEADS):
                w = plsc.load_gather(exbuf, [row_i, jnp.full((16,), g, jnp.int32)])
                hv = hbuf[e, pl.ds(g * HIDDEN, HIDDEN)]
                hbuf[e, pl.ds(g * HIDDEN, HIDDEN)] = hv * w
            return carry2

        lax.fori_loop(0, CHUNK, scale_body, 0)

        pltpu.sync_copy(hbuf, accm.at[dstv], add=True)
        pltpu.sync_copy(exbuf, accd.at[dstv], add=True)
        return carry

    lax.fori_loop(0, nq, chunk_body, 0)

    plsc.subcore_barrier()
    pltpu.sync_copy(accm.at[pl.ds(r0, ROWS_PER_SUB)],
                    outm_hbm.at[c, pl.ds(r0, ROWS_PER_SUB)])
    pltpu.sync_copy(accd.at[pl.ds(r0, ROWS_PER_SUB)],
                    outd_hbm.at[c, pl.ds(r0, ROWS_PER_SUB)])


_sc_edge = functools.partial(
    pl.kernel,
    out_type=(
        jax.ShapeDtypeStruct((NC, NPAD, F), jnp.float32),
        jax.ShapeDtypeStruct((NC, NPAD, AW), jnp.float32),
    ),
    mesh=_SC_MESH,
    compiler_params=pltpu.CompilerParams(
        needs_layout_passes=False, use_tc_tiling_on_sc=False),
    scratch_types=[
        pltpu.VMEM((CHUNK,), jnp.int32),
        pltpu.VMEM((CHUNK,), jnp.int32),
        pltpu.VMEM((CHUNK, F), jnp.float32),
        pltpu.VMEM((CHUNK, AW), jnp.float32),
        pltpu.VMEM((CHUNK, AW), jnp.float32),
        pltpu.VMEM((CHUNK, AW), jnp.float32),
        pltpu.VMEM_SHARED((NPAD, F), jnp.float32),
        pltpu.VMEM_SHARED((NPAD, AW), jnp.float32),
        pltpu.SemaphoreType.DMA,
        pltpu.SemaphoreType.DMA,
        pltpu.SemaphoreType.DMA,
        pltpu.SemaphoreType.DMA,
        pltpu.SemaphoreType.DMA,
    ],
)(_edge_body)


_BLK = 1000
_GRID = N // _BLK


def _tables_body(x_ref, wh_ref, was_ref, wad_ref, h_ref, as_ref, ad_ref):
    x = x_ref[...]
    h_ref[...] = jnp.dot(x, wh_ref[...], preferred_element_type=jnp.float32)
    as_ref[...] = jnp.dot(x, was_ref[...], preferred_element_type=jnp.float32)
    ad_ref[...] = jnp.dot(x, wad_ref[...], preferred_element_type=jnp.float32)


def _tc_tables(x, wh, was, wad):
    return pl.pallas_call(
        _tables_body,
        grid=(_GRID,),
        in_specs=[
            pl.BlockSpec((_BLK, F), lambda i: (i, 0)),
            pl.BlockSpec((F, F), lambda i: (0, 0)),
            pl.BlockSpec((F, AW), lambda i: (0, 0)),
            pl.BlockSpec((F, AW), lambda i: (0, 0)),
        ],
        out_specs=[
            pl.BlockSpec((_BLK, F), lambda i: (i, 0)),
            pl.BlockSpec((_BLK, AW), lambda i: (i, 0)),
            pl.BlockSpec((_BLK, AW), lambda i: (i, 0)),
        ],
        out_shape=[
            jax.ShapeDtypeStruct((N, F), jnp.float32),
            jax.ShapeDtypeStruct((N, AW), jnp.float32),
            jax.ShapeDtypeStruct((N, AW), jnp.float32),
        ],
    )(x, wh, was, wad)


def _finalize(pm_ref, pd_ref, b_ref, r_ref):
    m = pm_ref[0] + pm_ref[1]
    dsum = pd_ref[0] + pd_ref[1]
    rec = 1.0 / (dsum + 1e-16)
    recb = jnp.dot(rec, r_ref[...], preferred_element_type=jnp.float32)
    h = m * recb + b_ref[...]
    return jnp.where(h > 0, h, jnp.exp(h) - 1.0)


def _combine_body(pm_ref, pd_ref, b_ref, r_ref, wh_ref, was_ref, wad_ref,
                  h_ref, as_ref, ad_ref):
    act = _finalize(pm_ref, pd_ref, b_ref, r_ref)
    h_ref[...] = jnp.dot(act, wh_ref[...], preferred_element_type=jnp.float32)
    as_ref[...] = jnp.dot(act, was_ref[...], preferred_element_type=jnp.float32)
    ad_ref[...] = jnp.dot(act, wad_ref[...], preferred_element_type=jnp.float32)


def _tc_combine(pm, pd, b, r, wh, was, wad):
    return pl.pallas_call(
        _combine_body,
        grid=(_GRID,),
        in_specs=[
            pl.BlockSpec((NC, _BLK, F), lambda i: (0, i, 0)),
            pl.BlockSpec((NC, _BLK, AW), lambda i: (0, i, 0)),
            pl.BlockSpec((1, F), lambda i: (0, 0)),
            pl.BlockSpec((AW, F), lambda i: (0, 0)),
            pl.BlockSpec((F, F), lambda i: (0, 0)),
            pl.BlockSpec((F, AW), lambda i: (0, 0)),
            pl.BlockSpec((F, AW), lambda i: (0, 0)),
        ],
        out_specs=[
            pl.BlockSpec((_BLK, F), lambda i: (i, 0)),
            pl.BlockSpec((_BLK, AW), lambda i: (i, 0)),
            pl.BlockSpec((_BLK, AW), lambda i: (i, 0)),
        ],
        out_shape=[
            jax.ShapeDtypeStruct((N, F), jnp.float32),
            jax.ShapeDtypeStruct((N, AW), jnp.float32),
            jax.ShapeDtypeStruct((N, AW), jnp.float32),
        ],
    )(pm, pd, b, r, wh, was, wad)


def _head_body(pm_ref, pd_ref, b_ref, r_ref, lw_ref, extra_ref, out_ref):
    act = _finalize(pm_ref, pd_ref, b_ref, r_ref)
    colsum = jnp.sum(act[:N], axis=0)
    val = jnp.sum(colsum * lw_ref[0]) * (1.0 / N) + extra_ref[0, 0]
    out_ref[...] = jnp.reshape(val, (1, 1))


def _tc_head(pm, pd, b, r, lw, extra):
    return pl.pallas_call(
        _head_body,
        out_shape=jax.ShapeDtypeStruct((1, 1), jnp.float32),
    )(pm, pd, b, r, lw, extra)


def _fold_att(att):
    m = jnp.zeros((F, AW), jnp.float32)
    return m.at[jnp.arange(F), jnp.arange(F) // HIDDEN].set(att.reshape(F))


def kernel(x, edge_index, u, w, W1, att_src1, att_dst1, b1,
           W2, att_src2, att_dst2, b2, lin_W, lin_b):
    src = edge_index[0]
    dst = edge_index[1]

    was1 = W1 @ _fold_att(att_src1)
    wad1 = W1 @ _fold_att(att_dst1)
    was2 = W2 @ _fold_att(att_src2)
    wad2 = W2 @ _fold_att(att_dst2)
    rmat = (jnp.arange(F)[None, :] // HIDDEN
            == jnp.arange(AW)[:, None]).astype(jnp.float32)
    zm = jnp.zeros((NPAD, F), jnp.float32)
    zd = jnp.zeros((NPAD, AW), jnp.float32)
    b1r = b1.reshape(1, F)
    b2r = b2.reshape(1, F)
    lw = lin_W[:F, 0].reshape(1, F)
    extra = (u * lin_W[F, 0] + w * lin_W[F + 1, 0] + lin_b[0]).astype(
        jnp.float32).reshape(1, 1)

    h1, as1, ad1 = _tc_tables(x, W1, was1, wad1)
    pm1, pd1 = _sc_edge(src, dst, h1, as1, ad1, zm, zd)
    h2, as2, ad2 = _tc_combine(pm1, pd1, b1r, rmat, W2, was2, wad2)
    pm2, pd2 = _sc_edge(src, dst, h2, as2, ad2, zm, zd)
    out = _tc_head(pm2, pd2, b2r, rmat, lw, extra)
    return out.reshape(1)


# speedup vs baseline: 128.0254x; 2.2554x over previous
import functools

import jax
import jax.numpy as jnp
from jax import lax
from jax.experimental import pallas as pl
from jax.experimental.pallas import tpu as pltpu
from jax.experimental.pallas import tpu_sc as plsc

N = 10000
E = 320000
F = 128
HEADS = 8
HIDDEN = 16
AW = 16

NC, NS = 2, 16
NW = NC * NS
CHUNK = 80
QPW = E // (NW * CHUNK)
NPAD = 10112
ROWS_PER_SUB = NPAD // NS
NBUF = 3

_SC_MESH = plsc.VectorSubcoreMesh(
    core_axis_name="c", subcore_axis_name="s", num_cores=NC, num_subcores=NS
)


def _edge_body(src_hbm, dst_hbm, h_hbm, as_hbm, ad_hbm, zm_hbm, zd_hbm,
               outm_hbm, outd_hbm,
               srcbufs, dstbufs, scatbufs, hbufs, adbufs, exbufs,
               accm, accd, isems, gsems, ssems):
    c = lax.axis_index("c")
    s = lax.axis_index("s")
    gwid = s * NC + c

    r0 = s * ROWS_PER_SUB
    pltpu.sync_copy(zm_hbm.at[pl.ds(r0, ROWS_PER_SUB)],
                    accm.at[pl.ds(r0, ROWS_PER_SUB)])
    pltpu.sync_copy(zd_hbm.at[pl.ds(r0, ROWS_PER_SUB)],
                    accd.at[pl.ds(r0, ROWS_PER_SUB)])
    plsc.subcore_barrier()

    q0 = gwid * QPW

    def idx_copies(t, b):
        return (
            pltpu.make_async_copy(src_hbm.at[q0 + t], srcbufs[b], isems[b][0]),
            pltpu.make_async_copy(dst_hbm.at[q0 + t], dstbufs[b], isems[b][1]),
        )

    def gathers(t, b):
        return (
            pltpu.make_async_copy(h_hbm.at[srcbufs[b]], hbufs[b], gsems[b][0]),
            pltpu.make_async_copy(as_hbm.at[srcbufs[b]], exbufs[b], gsems[b][1]),
            pltpu.make_async_copy(ad_hbm.at[dstbufs[b]], adbufs[b], gsems[b][2]),
        )

    def scatters(b):
        return (
            pltpu.make_async_copy(hbufs[b], accm.at[scatbufs[b]], ssems[b][0]),
            pltpu.make_async_copy(exbufs[b], accd.at[scatbufs[b]], ssems[b][1]),
        )

    def compute_scatter(b):
        hbuf, adbuf, exbuf = hbufs[b], adbufs[b], exbufs[b]
        dstbuf, scatbuf = dstbufs[b], scatbufs[b]

        for k in range(CHUNK // 16):
            scatbuf[pl.ds(k * 16, 16)] = dstbuf[pl.ds(k * 16, 16)]

        dnums = lax.GatherDimensionNumbers(
            offset_dims=(), collapsed_slice_dims=(0,), start_index_map=(0,))

        def edge(e, carry):
            a = exbuf[e, :] + adbuf[e, :]
            ex = jnp.exp(jnp.maximum(a, 0.2 * a))
            exbuf[e, :] = ex
            for g in range(HEADS):
                wg = lax.gather(
                    ex, jnp.full((16, 1), g, jnp.int32), dnums, (1,),
                    mode=lax.GatherScatterMode.PROMISE_IN_BOUNDS)
                hv = hbuf[e, pl.ds(g * HIDDEN, HIDDEN)]
                hbuf[e, pl.ds(g * HIDDEN, HIDDEN)] = hv * wg
            return carry

        lax.fori_loop(0, CHUNK, edge, 0, unroll=2)
        cpm, cpd = scatters(b)
        cpm.start(add=True)
        cpd.start(add=True)

    def phase(t, cur, nxt, prv, ws=True, ii=True, ig=True):
        if ws:
            for cp in scatters(nxt):
                cp.wait()
        if ii:
            for cp in idx_copies(t + 2, prv):
                cp.start()
        if ig:
            for cp in idx_copies(t + 1, nxt):
                cp.wait()
            for cp in gathers(t + 1, nxt):
                cp.start()
        for cp in gathers(t, cur):
            cp.wait()
        compute_scatter(cur)

    for cp in idx_copies(0, 0):
        cp.start()
    for cp in idx_copies(1, 1):
        cp.start()
    for cp in idx_copies(0, 0):
        cp.wait()
    for cp in gathers(0, 0):
        cp.start()
    for cp in idx_copies(2, 2):
        cp.start()
    for cp in idx_copies(1, 1):
        cp.wait()
    for cp in gathers(1, 1):
        cp.start()
    for cp in gathers(0, 0):
        cp.wait()
    compute_scatter(0)
    phase(1, 1, 2, 0, ws=False)

    def triple(p, carry):
        t = 3 * p + 2
        phase(t, 2, 0, 1)
        phase(t + 1, 0, 1, 2)
        phase(t + 2, 1, 2, 0)
        return carry

    lax.fori_loop(0, (QPW - 5) // NBUF, triple, 0)
    phase(QPW - 3, 2, 0, 1)
    phase(QPW - 2, 0, 1, 2, ii=False)
    phase(QPW - 1, 1, 2, 0, ii=False, ig=False)
    for cp in scatters(0):
        cp.wait()
    for cp in scatters(1):
        cp.wait()

    plsc.subcore_barrier()
    pltpu.sync_copy(accm.at[pl.ds(r0, ROWS_PER_SUB)],
                    outm_hbm.at[c, pl.ds(r0, ROWS_PER_SUB)])
    pltpu.sync_copy(accd.at[pl.ds(r0, ROWS_PER_SUB)],
                    outd_hbm.at[c, pl.ds(r0, ROWS_PER_SUB)])


_sc_edge = functools.partial(
    pl.kernel,
    out_type=(
        jax.ShapeDtypeStruct((NC, NPAD, F), jnp.float32),
        jax.ShapeDtypeStruct((NC, NPAD, AW), jnp.float32),
    ),
    mesh=_SC_MESH,
    compiler_params=pltpu.CompilerParams(
        needs_layout_passes=False, use_tc_tiling_on_sc=False),
    scratch_types=[
        [pltpu.VMEM((CHUNK,), jnp.int32)] * NBUF,
        [pltpu.VMEM((CHUNK,), jnp.int32)] * NBUF,
        [pltpu.VMEM((CHUNK,), jnp.int32)] * NBUF,
        [pltpu.VMEM((CHUNK, F), jnp.float32)] * NBUF,
        [pltpu.VMEM((CHUNK, AW), jnp.float32)] * NBUF,
        [pltpu.VMEM((CHUNK, AW), jnp.float32)] * NBUF,
        pltpu.VMEM_SHARED((NPAD, F), jnp.float32),
        pltpu.VMEM_SHARED((NPAD, AW), jnp.float32),
        [[pltpu.SemaphoreType.DMA] * 2] * NBUF,
        [[pltpu.SemaphoreType.DMA] * 3] * NBUF,
        [[pltpu.SemaphoreType.DMA] * 2] * NBUF,
    ],
)(_edge_body)


_BLK = 1000
_GRID = N // _BLK


def _tables_body(x_ref, wh_ref, was_ref, wad_ref, rt_ref, h_ref, as_ref, ad_ref):
    h = jnp.dot(x_ref[...], wh_ref[...], preferred_element_type=jnp.float32)
    h_ref[...] = h
    as_ref[...] = jnp.dot(h * was_ref[...], rt_ref[...],
                          preferred_element_type=jnp.float32,
                          precision=lax.Precision.HIGHEST)
    ad_ref[...] = jnp.dot(h * wad_ref[...], rt_ref[...],
                          preferred_element_type=jnp.float32,
                          precision=lax.Precision.HIGHEST)


def _tc_tables(x, wh, was, wad, rt):
    return pl.pallas_call(
        _tables_body,
        grid=(_GRID,),
        in_specs=[
            pl.BlockSpec((_BLK, F), lambda i: (i, 0)),
            pl.BlockSpec((F, F), lambda i: (0, 0)),
            pl.BlockSpec((1, F), lambda i: (0, 0)),
            pl.BlockSpec((1, F), lambda i: (0, 0)),
            pl.BlockSpec((F, AW), lambda i: (0, 0)),
        ],
        out_specs=[
            pl.BlockSpec((_BLK, F), lambda i: (i, 0)),
            pl.BlockSpec((_BLK, AW), lambda i: (i, 0)),
            pl.BlockSpec((_BLK, AW), lambda i: (i, 0)),
        ],
        out_shape=[
            jax.ShapeDtypeStruct((N, F), jnp.float32),
            jax.ShapeDtypeStruct((N, AW), jnp.float32),
            jax.ShapeDtypeStruct((N, AW), jnp.float32),
        ],
    )(x, wh, was, wad, rt)


def _finalize(pm_ref, pd_ref, b_ref, r_ref):
    m = pm_ref[0] + pm_ref[1]
    dsum = pd_ref[0] + pd_ref[1]
    rec = 1.0 / (dsum + 1e-16)
    recb = jnp.dot(rec, r_ref[...], preferred_element_type=jnp.float32,
                   precision=lax.Precision.HIGHEST)
    h = m * recb + b_ref[...]
    return jnp.where(h > 0, h, jnp.exp(h) - 1.0)


def _combine_body(pm_ref, pd_ref, b_ref, r_ref, wh_ref, was_ref, wad_ref,
                  rt_ref, h_ref, as_ref, ad_ref):
    act = _finalize(pm_ref, pd_ref, b_ref, r_ref)
    h = jnp.dot(act, wh_ref[...], preferred_element_type=jnp.float32)
    h_ref[...] = h
    as_ref[...] = jnp.dot(h * was_ref[...], rt_ref[...],
                          preferred_element_type=jnp.float32,
                          precision=lax.Precision.HIGHEST)
    ad_ref[...] = jnp.dot(h * wad_ref[...], rt_ref[...],
                          preferred_element_type=jnp.float32,
                          precision=lax.Precision.HIGHEST)


def _tc_combine(pm, pd, b, r, wh, was, wad, rt):
    return pl.pallas_call(
        _combine_body,
        grid=(_GRID,),
        in_specs=[
            pl.BlockSpec((NC, _BLK, F), lambda i: (0, i, 0)),
            pl.BlockSpec((NC, _BLK, AW), lambda i: (0, i, 0)),
            pl.BlockSpec((1, F), lambda i: (0, 0)),
            pl.BlockSpec((AW, F), lambda i: (0, 0)),
            pl.BlockSpec((F, F), lambda i: (0, 0)),
            pl.BlockSpec((1, F), lambda i: (0, 0)),
            pl.BlockSpec((1, F), lambda i: (0, 0)),
            pl.BlockSpec((F, AW), lambda i: (0, 0)),
        ],
        out_specs=[
            pl.BlockSpec((_BLK, F), lambda i: (i, 0)),
            pl.BlockSpec((_BLK, AW), lambda i: (i, 0)),
            pl.BlockSpec((_BLK, AW), lambda i: (i, 0)),
        ],
        out_shape=[
            jax.ShapeDtypeStruct((N, F), jnp.float32),
            jax.ShapeDtypeStruct((N, AW), jnp.float32),
            jax.ShapeDtypeStruct((N, AW), jnp.float32),
        ],
    )(pm, pd, b, r, wh, was, wad, rt)


def _head_body(pm_ref, pd_ref, b_ref, r_ref, lw_ref, extra_ref, out_ref):
    act = _finalize(pm_ref, pd_ref, b_ref, r_ref)
    r = act[:N]
    for _ in range(4):
        m = r.shape[0] // 2
        r = r[:m] + r[m:]
    for _ in range(4):
        m = r.shape[0] // 5
        r = (r[:m] + r[m:2 * m]) + (r[2 * m:3 * m] + r[3 * m:4 * m]) + r[4 * m:]
    colsum = r[0]
    val = jnp.sum(colsum * lw_ref[0]) * (1.0 / N) + extra_ref[0, 0]
    out_ref[...] = jnp.reshape(val, (1, 1))


def _tc_head(pm, pd, b, r, lw, extra):
    return pl.pallas_call(
        _head_body,
        out_shape=jax.ShapeDtypeStruct((1, 1), jnp.float32),
    )(pm, pd, b, r, lw, extra)


def kernel(x, edge_index, u, w, W1, att_src1, att_dst1, b1,
           W2, att_src2, att_dst2, b2, lin_W, lin_b):
    src = edge_index[0].reshape(-1, CHUNK)
    dst = edge_index[1].reshape(-1, CHUNK)

    was1 = att_src1.reshape(1, F)
    wad1 = att_dst1.reshape(1, F)
    was2 = att_src2.reshape(1, F)
    wad2 = att_dst2.reshape(1, F)
    rmat = (jnp.arange(F)[None, :] // HIDDEN
            == jnp.arange(AW)[:, None]).astype(jnp.float32)
    rtmat = rmat.T
    zm = jnp.zeros((NPAD, F), jnp.float32)
    zd = jnp.zeros((NPAD, AW), jnp.float32)
    b1r = b1.reshape(1, F)
    b2r = b2.reshape(1, F)
    lw = lin_W[:F, 0].reshape(1, F)
    extra = (u * lin_W[F, 0] + w * lin_W[F + 1, 0] + lin_b[0]).astype(
        jnp.float32).reshape(1, 1)

    h1, as1, ad1 = _tc_tables(x, W1, was1, wad1, rtmat)
    pm1, pd1 = _sc_edge(src, dst, h1, as1, ad1, zm, zd)
    h2, as2, ad2 = _tc_combine(pm1, pd1, b1r, rmat, W2, was2, wad2, rtmat)
    pm2, pd2 = _sc_edge(src, dst, h2, as2, ad2, zm, zd)
    out = _tc_head(pm2, pd2, b2r, rmat, lw, extra)
    return out.reshape(1)
